# baseline (speedup 1.0000x reference)
import jax
import jax.numpy as jnp
from jax import lax
from jax.experimental import pallas as pl
from jax.experimental.pallas import tpu as pltpu
from jax.experimental.pallas import tpu_sc as plsc

LEADS = 16384
L = 4096
HALF = 2048
NC = 2
NS = 16
NW = NC * NS
RPW = LEADS // NW
G = 16
NG = RPW // G
W = 128
MID = HALF - W


def _row_params(s):
    t = s & (HALF - 1)
    a = pl.multiple_of(t & (-W), W)
    rmd = t - a
    swap = s < HALF
    return a, rmd, swap


def _ladder(x_hbm, o_hbm, zeros_v, row, seg_off, seg_len, swap, sem):
    off = seg_off
    for k in (3, 2, 1, 0):
        size = W << k
        bit = (seg_len & size) != 0
        offk = pl.multiple_of(off, W)

        @pl.when(bit & swap)
        def _():
            pltpu.make_async_copy(
                x_hbm.at[row, pl.ds(offk, size)],
                o_hbm.at[row, pl.ds(offk, size)], sem).start()

        @pl.when(bit & jnp.logical_not(swap))
        def _():
            pltpu.make_async_copy(
                zeros_v.at[pl.ds(0, size)],
                o_hbm.at[row, pl.ds(offk, size)], sem).start()

        off = off + jnp.where(bit, size, 0)


def _sc_body(x_hbm, st_hbm, o_hbm, st_v, zeros_v, wr_v, ww_v, sem_rd,
             sem_body):
    wid = lax.axis_index("s") * NC + lax.axis_index("c")
    base = wid * RPW
    pltpu.sync_copy(st_hbm.at[pl.ds(base, RPW)], st_v.at[pl.ds(0, RPW)])

    z16 = jnp.zeros((16,), jnp.float32)

    def zf(j, _):
        zeros_v[pl.ds(j * 16, 16)] = z16
        return 0
    lax.fori_loop(0, MID // 16, zf, 0)

    jv = lax.broadcasted_iota(jnp.int32, (16,), 0)

    def group(g, _):
        grow = base + g * G
        wslot = (g & 1) * (G * 2 * W)

        @pl.when(g >= 2)
        def _():
            pltpu.make_async_copy(
                x_hbm.at[pl.ds(0, G), :], o_hbm.at[pl.ds(0, G), :],
                sem_body).wait()

        def issue(i, _):
            s = st_v[pl.ds(g * G + i, 16)][0]
            a, rmd, swap = _row_params(s)
            row = grow + i
            pltpu.make_async_copy(
                x_hbm.at[row, pl.ds(a, W)],
                wr_v.at[pl.ds(i * 2 * W, W)], sem_rd).start()
            pltpu.make_async_copy(
                x_hbm.at[row, pl.ds(a + HALF, W)],
                wr_v.at[pl.ds(i * 2 * W + W, W)], sem_rd).start()
            amid = pl.multiple_of(a + W, W)

            @pl.when(swap)
            def _():
                pltpu.make_async_copy(
                    zeros_v.at[pl.ds(0, MID)],
                    o_hbm.at[row, pl.ds(amid, MID)], sem_body).start()

            @pl.when(jnp.logical_not(swap))
            def _():
                pltpu.make_async_copy(
                    x_hbm.at[row, pl.ds(amid, MID)],
                    o_hbm.at[row, pl.ds(amid, MID)], sem_body).start()

            _ladder(x_hbm, o_hbm, zeros_v, row, 0, a, swap, sem_body)
            _ladder(x_hbm, o_hbm, zeros_v, row, a + HALF + W, MID - a, swap,
                    sem_body)
            return 0
        lax.fori_loop(0, G, issue, 0)

        pltpu.make_async_copy(
            x_hbm.at[0, :], wr_v.at[pl.ds(0, G * 2 * W)], sem_rd).wait()

        def blend(i, _):
            s = st_v[pl.ds(g * G + i, 16)][0]
            a, rmd, swap = _row_params(s)
            row = grow + i
            lo1 = jnp.where(swap, 0, rmd)
            hi1 = jnp.where(swap, rmd, W)
            lo2 = jnp.where(swap, rmd, 0)
            hi2 = jnp.where(swap, W, rmd)
            for sub in range(W // 16):
                jf = jv + sub * 16
                r_off = pl.multiple_of(i * 2 * W + sub * 16, 16)
                w_off = pl.multiple_of(wslot + i * 2 * W + sub * 16, 16)
                w1 = wr_v[pl.ds(r_off, 16)]
                ww_v[pl.ds(w_off, 16)] = jnp.where(
                    (jf >= lo1) & (jf < hi1), w1, 0.0)
                w2 = wr_v[pl.ds(r_off + W, 16)]
                ww_v[pl.ds(w_off + W, 16)] = jnp.where(
                    (jf >= lo2) & (jf < hi2), w2, 0.0)
            pltpu.make_async_copy(
                ww_v.at[pl.ds(wslot + i * 2 * W, W)],
                o_hbm.at[row, pl.ds(a, W)], sem_body).start()
            pltpu.make_async_copy(
                ww_v.at[pl.ds(wslot + i * 2 * W + W, W)],
                o_hbm.at[row, pl.ds(a + HALF, W)], sem_body).start()
            return 0
        lax.fori_loop(0, G, blend, 0)
        return 0

    lax.fori_loop(0, NG, group, 0)

    pltpu.make_async_copy(
        x_hbm.at[pl.ds(0, G), :], o_hbm.at[pl.ds(0, G), :], sem_body).wait()
    pltpu.make_async_copy(
        x_hbm.at[pl.ds(0, G), :], o_hbm.at[pl.ds(0, G), :], sem_body).wait()


def kernel(x, starts):
    out = pl.kernel(
        _sc_body,
        out_type=jax.ShapeDtypeStruct((LEADS, L), jnp.float32),
        mesh=plsc.VectorSubcoreMesh(core_axis_name="c", subcore_axis_name="s"),
        scratch_types=[
            pltpu.VMEM((RPW + 16,), jnp.int32),
            pltpu.VMEM((MID,), jnp.float32),
            pltpu.VMEM((G * 2 * W,), jnp.float32),
            pltpu.VMEM((2 * G * 2 * W,), jnp.float32),
            pltpu.SemaphoreType.DMA,
            pltpu.SemaphoreType.DMA,
        ],
    )(x, starts)
    return out


# speedup vs baseline: 16.4995x; 16.4995x over previous
import jax
import jax.numpy as jnp
from jax import lax
from jax.experimental import pallas as pl
from jax.experimental.pallas import tpu as pltpu
from jax.experimental.pallas import tpu_sc as plsc

LEADS = 16384
L = 4096
HALF = 2048
NC = 2
NS = 16
NW = NC * NS
RPW = LEADS // NW
RB = 8
NBATCH = RPW // RB
SLOT = RB * L

def _fix_rows(st_v, buf_v, rbase, soff):
    jv = lax.broadcasted_iota(jnp.int32, (16,), 0)
    z16 = jnp.zeros((16,), jnp.float32)

    def row(i, _):
        s = st_v[pl.ds(rbase + i, 16)][0]
        ap = s & (-16)
        rmd = s - ap
        br = soff + i
        for j in range(1, HALF // 16):
            off = pl.multiple_of((ap + 16 * j) & (L - 1), 16)
            buf_v[br, pl.ds(off, 16)] = z16
        o1 = pl.multiple_of(ap & (L - 1), 16)
        w1 = buf_v[br, pl.ds(o1, 16)]
        buf_v[br, pl.ds(o1, 16)] = jnp.where(jv < rmd, w1, 0.0)
        o2 = pl.multiple_of((ap + HALF) & (L - 1), 16)
        w2 = buf_v[br, pl.ds(o2, 16)]
        buf_v[br, pl.ds(o2, 16)] = jnp.where(jv >= rmd, w2, 0.0)
        return 0

    lax.fori_loop(0, RB, row, 0)


def _sc_body(x_hbm, st_hbm, o_hbm, st_v, buf_v, sem_r0, sem_r1,
             sem_w0, sem_w1):
    wid = lax.axis_index("s") * NC + lax.axis_index("c")
    base = wid * RPW
    pltpu.sync_copy(st_hbm.at[pl.ds(base, RPW)], st_v.at[pl.ds(0, RPW)])

    def rd(bi, soff, sem):
        return pltpu.make_async_copy(
            x_hbm.at[pl.ds(base + bi * RB, RB), :],
            buf_v.at[pl.ds(soff, RB), :], sem)

    def wr(bi, soff, sem):
        return pltpu.make_async_copy(
            buf_v.at[pl.ds(soff, RB), :],
            o_hbm.at[pl.ds(base + bi * RB, RB), :], sem)

    def pair(p, _):
        b0, b1 = 2 * p, 2 * p + 1

        @pl.when(p > 0)
        def _():
            wr(b0 - 2, 0, sem_w0).wait()
        rd(b0, 0, sem_r0).start()

        @pl.when(p > 0)
        def _():
            wr(b1 - 2, RB, sem_w1).wait()
        rd(b1, RB, sem_r1).start()

        rd(b0, 0, sem_r0).wait()
        _fix_rows(st_v, buf_v, b0 * RB, 0)
        wr(b0, 0, sem_w0).start()

        rd(b1, RB, sem_r1).wait()
        _fix_rows(st_v, buf_v, b1 * RB, RB)
        wr(b1, RB, sem_w1).start()
        return 0

    lax.fori_loop(0, NBATCH // 2, pair, 0)
    wr(NBATCH - 2, 0, sem_w0).wait()
    wr(NBATCH - 1, RB, sem_w1).wait()


def kernel(x, starts):
    out = pl.kernel(
        _sc_body,
        out_type=jax.ShapeDtypeStruct((LEADS, L), jnp.float32),
        mesh=plsc.VectorSubcoreMesh(core_axis_name="c", subcore_axis_name="s"),
        scratch_types=[
            pltpu.VMEM((RPW + 16,), jnp.int32),
            pltpu.VMEM((2 * RB, L), jnp.float32),
            pltpu.SemaphoreType.DMA,
            pltpu.SemaphoreType.DMA,
            pltpu.SemaphoreType.DMA,
            pltpu.SemaphoreType.DMA,
        ],
    )(x, starts)
    return out


# speedup vs baseline: 16.6758x; 1.0107x over previous
import jax
import jax.numpy as jnp
from jax import lax
from jax.experimental import pallas as pl
from jax.experimental.pallas import tpu as pltpu
from jax.experimental.pallas import tpu_sc as plsc

LEADS = 16384
L = 4096
HALF = 2048
NC = 2
NS = 16
NW = NC * NS
RPW = LEADS // NW
RB = 8
NBATCH = RPW // RB
SLOT = RB * L

def _fix_rows(st_v, buf_v, rbase, soff):
    jv = lax.broadcasted_iota(jnp.int32, (16,), 0)
    z16 = jnp.zeros((16,), jnp.float32)

    def row(i, _):
        s = st_v[pl.ds(rbase + i, 16)][0]
        ap = s & (-16)
        rmd = s - ap
        br = soff + i
        for j in range(1, HALF // 16):
            off = pl.multiple_of((ap + 16 * j) & (L - 1), 16)
            buf_v[br, pl.ds(off, 16)] = z16
        o1 = pl.multiple_of(ap & (L - 1), 16)
        w1 = buf_v[br, pl.ds(o1, 16)]
        buf_v[br, pl.ds(o1, 16)] = jnp.where(jv < rmd, w1, 0.0)
        o2 = pl.multiple_of((ap + HALF) & (L - 1), 16)
        w2 = buf_v[br, pl.ds(o2, 16)]
        buf_v[br, pl.ds(o2, 16)] = jnp.where(jv >= rmd, w2, 0.0)
        return 0

    lax.fori_loop(0, RB, row, 0)


def _sc_body(x_hbm, st_hbm, o_hbm, st_v, buf_v, sem_r0, sem_r1, sem_r2,
             sem_w0, sem_w1, sem_w2):
    wid = lax.axis_index("s") * NC + lax.axis_index("c")
    base = wid * RPW
    pltpu.sync_copy(st_hbm.at[pl.ds(base, RPW)], st_v.at[pl.ds(0, RPW)])

    sem_r = (sem_r0, sem_r1, sem_r2)
    sem_w = (sem_w0, sem_w1, sem_w2)

    def rd(bi, slot):
        return pltpu.make_async_copy(
            x_hbm.at[pl.ds(base + bi * RB, RB), :],
            buf_v.at[pl.ds(slot * RB, RB), :], sem_r[slot])

    def wr(bi, slot):
        return pltpu.make_async_copy(
            buf_v.at[pl.ds(slot * RB, RB), :],
            o_hbm.at[pl.ds(base + bi * RB, RB), :], sem_w[slot])

    rd(0, 0).start()
    rd(1, 1).start()

    def triple(p, _):
        for k in range(3):
            b = 3 * p + k
            ks = (k + 2) % 3

            @pl.when(b + 2 < NBATCH)
            def _():
                @pl.when(b >= 1)
                def _():
                    wr(b - 1, ks).wait()
                rd(b + 2, ks).start()

            rd(b, k).wait()
            _fix_rows(st_v, buf_v, b * RB, k * RB)
            wr(b, k).start()
        return 0

    lax.fori_loop(0, (NBATCH - 1) // 3, triple, 0)

    rd(NBATCH - 1, 0).wait()
    _fix_rows(st_v, buf_v, (NBATCH - 1) * RB, 0)
    wr(NBATCH - 1, 0).start()

    wr(NBATCH - 3, 1).wait()
    wr(NBATCH - 2, 2).wait()
    wr(NBATCH - 1, 0).wait()


def kernel(x, starts):
    out = pl.kernel(
        _sc_body,
        out_type=jax.ShapeDtypeStruct((LEADS, L), jnp.float32),
        mesh=plsc.VectorSubcoreMesh(core_axis_name="c", subcore_axis_name="s"),
        scratch_types=[
            pltpu.VMEM((RPW + 16,), jnp.int32),
            pltpu.VMEM((3 * RB, L), jnp.float32),
            pltpu.SemaphoreType.DMA,
            pltpu.SemaphoreType.DMA,
            pltpu.SemaphoreType.DMA,
            pltpu.SemaphoreType.DMA,
            pltpu.SemaphoreType.DMA,
            pltpu.SemaphoreType.DMA,
        ],
    )(x, starts)
    return out


# speedup vs baseline: 17.7969x; 1.0672x over previous
import jax
import jax.numpy as jnp
from jax import lax
from jax.experimental import pallas as pl
from jax.experimental.pallas import tpu as pltpu
from jax.experimental.pallas import tpu_sc as plsc

LEADS = 16384
L = 4096
HALF = 2048
NC = 2
NS = 16
NW = NC * NS
RPW = LEADS // NW
RB = 8
NBATCH = RPW // RB
SLOT = RB * L

def _fix_rows(st_v, buf_v, rbase, soff, h):
    rbase = rbase + h * (RB // 2)
    soff = soff + h * (RB // 2)
    jv = lax.broadcasted_iota(jnp.int32, (16,), 0)
    z16 = jnp.zeros((16,), jnp.float32)

    def row(i, _):
        s = st_v[pl.ds(rbase + i, 16)][0]
        ap = s & (-16)
        rmd = s - ap
        br = soff + i
        for j in range(1, HALF // 16):
            off = pl.multiple_of((ap + 16 * j) & (L - 1), 16)
            buf_v[br, pl.ds(off, 16)] = z16
        o1 = pl.multiple_of(ap & (L - 1), 16)
        w1 = buf_v[br, pl.ds(o1, 16)]
        buf_v[br, pl.ds(o1, 16)] = jnp.where(jv < rmd, w1, 0.0)
        o2 = pl.multiple_of((ap + HALF) & (L - 1), 16)
        w2 = buf_v[br, pl.ds(o2, 16)]
        buf_v[br, pl.ds(o2, 16)] = jnp.where(jv >= rmd, w2, 0.0)
        return 0

    lax.fori_loop(0, RB // 2, row, 0)


def _sc_body(x_hbm, st_hbm, o_hbm, st_v, buf_v, sem_r0, sem_r1, sem_r2,
             sem_w0, sem_w1, sem_w2):
    wid = lax.axis_index("s") * NC + lax.axis_index("c")
    base = wid * RPW
    pltpu.sync_copy(st_hbm.at[pl.ds(base, RPW)], st_v.at[pl.ds(0, RPW)])

    sem_r = (sem_r0, sem_r1, sem_r2)
    sem_w = (sem_w0, sem_w1, sem_w2)

    def rd(bi, slot):
        return pltpu.make_async_copy(
            x_hbm.at[pl.ds(base + bi * RB, RB), :],
            buf_v.at[pl.ds(slot * RB, RB), :], sem_r[slot])

    def wr_half(bi, slot, h):
        hh = RB // 2
        return pltpu.make_async_copy(
            buf_v.at[pl.ds(slot * RB + h * hh, hh), :],
            o_hbm.at[pl.ds(base + bi * RB + h * hh, hh), :], sem_w[slot])

    rd(0, 0).start()
    rd(1, 1).start()

    def triple(p, _):
        for k in range(3):
            b = 3 * p + k
            ks = (k + 2) % 3

            @pl.when(b + 2 < NBATCH)
            def _():
                @pl.when(b >= 1)
                def _():
                    wr_half(b - 1, ks, 0).wait()
                    wr_half(b - 1, ks, 1).wait()
                rd(b + 2, ks).start()

            rd(b, k).wait()
            _fix_rows(st_v, buf_v, b * RB, k * RB, 0)
            wr_half(b, k, 0).start()
            _fix_rows(st_v, buf_v, b * RB, k * RB, 1)
            wr_half(b, k, 1).start()
        return 0

    lax.fori_loop(0, (NBATCH - 1) // 3, triple, 0)

    rd(NBATCH - 1, 0).wait()
    _fix_rows(st_v, buf_v, (NBATCH - 1) * RB, 0, 0)
    wr_half(NBATCH - 1, 0, 0).start()
    _fix_rows(st_v, buf_v, (NBATCH - 1) * RB, 0, 1)
    wr_half(NBATCH - 1, 0, 1).start()

    for bi, sl in ((NBATCH - 3, 1), (NBATCH - 2, 2), (NBATCH - 1, 0)):
        wr_half(bi, sl, 0).wait()
        wr_half(bi, sl, 1).wait()


def kernel(x, starts):
    out = pl.kernel(
        _sc_body,
        out_type=jax.ShapeDtypeStruct((LEADS, L), jnp.float32),
        mesh=plsc.VectorSubcoreMesh(core_axis_name="c", subcore_axis_name="s"),
        scratch_types=[
            pltpu.VMEM((RPW + 16,), jnp.int32),
            pltpu.VMEM((3 * RB, L), jnp.float32),
            pltpu.SemaphoreType.DMA,
            pltpu.SemaphoreType.DMA,
            pltpu.SemaphoreType.DMA,
            pltpu.SemaphoreType.DMA,
            pltpu.SemaphoreType.DMA,
            pltpu.SemaphoreType.DMA,
        ],
    )(x, starts)
    return out
